# baseline (speedup 1.0000x reference)
import jax
import jax.numpy as jnp
from jax import lax
from jax.experimental import pallas as pl
from jax.experimental.pallas import tpu as pltpu
from jax.experimental.pallas import tpu_sc as plsc

NUM_GRAPHS = 1024
HIDDEN = 14
N_NODES = 100000
W = 16
LANES = 16

NC = 2
NS = 16
NW = NC * NS
CHUNK = 3200
GB = 128
G = CHUNK // GB
N_PAD = NW * CHUNK
ROWS_PER_TILE = NUM_GRAPHS // NS
UNROLL = 4


def _seg_body(x_hbm, idx_hbm, z_hbm, out_p, xpack, xv, idxv, acc, sem):
    cid = lax.axis_index("c")
    sid = lax.axis_index("s")
    wid = cid * NS + sid
    base = wid * CHUNK
    last_real = N_NODES - (NW - 1) * CHUNK

    @pl.when(wid < NW - 1)
    def _():
        pltpu.async_copy(x_hbm.at[pl.ds(base * HIDDEN, CHUNK * HIDDEN)],
                         xpack.at[pl.ds(0, CHUNK * HIDDEN)], sem).wait()

    @pl.when(wid == NW - 1)
    def _():
        pltpu.async_copy(x_hbm.at[pl.ds(base * HIDDEN, last_real * HIDDEN)],
                         xpack.at[pl.ds(0, last_real * HIDDEN)], sem).wait()

    pltpu.sync_copy(z_hbm, acc.at[pl.ds(sid * ROWS_PER_TILE, ROWS_PER_TILE), :])
    for g in range(G):
        pltpu.sync_copy(idx_hbm.at[pl.ds(base + g * GB, GB)], idxv.at[g, :])

    plsc.subcore_barrier()

    lane = lax.broadcasted_iota(jnp.int32, (LANES,), 0)
    ones_lane = lane == HIDDEN

    def gbody(g, carry):
        def rbody(j, c2):
            for u in range(UNROLL):
                i = g * GB + j * UNROLL + u
                v = xpack[pl.ds(i * HIDDEN, LANES)]
                xv[i, :] = jnp.where(ones_lane, 1.0, v)
            return c2

        lax.fori_loop(0, GB // UNROLL, rbody, 0, unroll=2)
        pltpu.sync_copy(xv.at[pl.ds(g * GB, GB), :], acc.at[idxv.at[g, :]],
                        add=True)
        return carry

    lax.fori_loop(0, G, gbody, 0)
    plsc.subcore_barrier()

    pltpu.sync_copy(acc.at[pl.ds(sid * ROWS_PER_TILE, ROWS_PER_TILE), :],
                    out_p.at[cid, pl.ds(sid * ROWS_PER_TILE, ROWS_PER_TILE), :])


_seg_kernel = pl.kernel(
    _seg_body,
    out_type=jax.ShapeDtypeStruct((NC, NUM_GRAPHS, W), jnp.float32),
    mesh=plsc.VectorSubcoreMesh(core_axis_name="c", subcore_axis_name="s",
                                num_cores=NC, num_subcores=NS),
    scratch_types=[
        pltpu.VMEM((CHUNK * HIDDEN + LANES,), jnp.float32),
        pltpu.VMEM((CHUNK, W), jnp.float32),
        pltpu.VMEM((G, GB), jnp.int32),
        pltpu.VMEM_SHARED((NUM_GRAPHS + 1, W), jnp.float32),
        pltpu.SemaphoreType.DMA,
    ],
    compiler_params=pltpu.CompilerParams(use_tc_tiling_on_sc=False),
)


def _mlp_body(p, w1t, b1, w2t, b2, o):
    a = p[0] + p[1]
    cnt = jnp.maximum(a[:, HIDDEN:HIDDEN + 1], 1.0)
    mean = a[:, :HIDDEN] / cnt
    h = jnp.maximum(
        jnp.dot(mean, w1t[...], preferred_element_type=jnp.float32) + b1[...], 0.0)
    o[...] = jnp.dot(h, w2t[...], preferred_element_type=jnp.float32) + b2[...]


def _mlp(p, w1t, b1, w2t, b2):
    return pl.pallas_call(
        _mlp_body,
        out_shape=jax.ShapeDtypeStruct((NUM_GRAPHS, 2), jnp.float32),
    )(p, w1t, b1, w2t, b2)


def kernel(x, edge_index, edge_attr, u, batch, W1, b1, W2, b2):
    x_flat = x.reshape(N_NODES * HIDDEN)
    idx = jnp.pad(batch.astype(jnp.int32), (0, N_PAD - N_NODES),
                  constant_values=NUM_GRAPHS)
    z = jnp.zeros((ROWS_PER_TILE, W), jnp.float32)
    p = _seg_kernel(x_flat, idx, z)
    return _mlp(p, W1.T, b1[None, :], W2.T, b2[None, :])


# speedup vs baseline: 1.5776x; 1.5776x over previous
import jax
import jax.numpy as jnp
from jax import lax
from jax.experimental import pallas as pl
from jax.experimental.pallas import tpu as pltpu
from jax.experimental.pallas import tpu_sc as plsc

NUM_GRAPHS = 1024
HIDDEN = 14
N_NODES = 100000
W = 16
WOUT = 128

NC = 2
NS = 16
NW = NC * NS
CHUNK = 3200
GB = 128
G = CHUNK // GB
N_PAD = NW * CHUNK
ROWS_PER_TILE = NUM_GRAPHS // NS


def _seg_body(x_hbm, idx_hbm, z_hbm, out_p, xv, idxv, acc, sem):
    cid = lax.axis_index("c")
    sid = lax.axis_index("s")
    wid = cid * NS + sid
    base = wid * CHUNK
    last_real = N_NODES - (NW - 1) * CHUNK

    pend = [
        pltpu.async_copy(
            z_hbm, acc.at[pl.ds(sid * ROWS_PER_TILE, ROWS_PER_TILE), :], sem),
    ]
    pend += [
        pltpu.async_copy(idx_hbm.at[pl.ds(base + g * GB, GB)],
                         idxv.at[g, :], sem)
        for g in range(G)
    ]

    @pl.when(wid < NW - 1)
    def _():
        pltpu.async_copy(x_hbm.at[pl.ds(base, CHUNK), :],
                         xv.at[pl.ds(0, CHUNK), :], sem).wait()

    @pl.when(wid == NW - 1)
    def _():
        pltpu.async_copy(x_hbm.at[pl.ds(base, last_real), :],
                         xv.at[pl.ds(0, last_real), :], sem).wait()

    for h in pend:
        h.wait()

    plsc.subcore_barrier()

    scat = [
        pltpu.async_copy(xv.at[pl.ds(g * GB, GB), :], acc.at[idxv.at[g, :]],
                         sem, add=True)
        for g in range(G)
    ]
    for h in scat:
        h.wait()

    plsc.subcore_barrier()

    pltpu.sync_copy(acc.at[pl.ds(sid * ROWS_PER_TILE, ROWS_PER_TILE), :],
                    out_p.at[cid, pl.ds(sid * ROWS_PER_TILE, ROWS_PER_TILE),
                             pl.ds(0, W)])


_seg_kernel = pl.kernel(
    _seg_body,
    out_type=jax.ShapeDtypeStruct((NC, NUM_GRAPHS, WOUT), jnp.float32),
    mesh=plsc.VectorSubcoreMesh(core_axis_name="c", subcore_axis_name="s",
                                num_cores=NC, num_subcores=NS),
    scratch_types=[
        pltpu.VMEM((CHUNK, W), jnp.float32),
        pltpu.VMEM((G, GB), jnp.int32),
        pltpu.VMEM_SHARED((NUM_GRAPHS + 1, W), jnp.float32),
        pltpu.SemaphoreType.DMA,
    ],
    compiler_params=pltpu.CompilerParams(use_tc_tiling_on_sc=False),
)


def _mlp_body(p, w1t, b1, w2t, b2, o):
    a = p[0] + p[1]
    cnt = jnp.maximum(a[:, HIDDEN:HIDDEN + 1], 1.0)
    mean = a[:, :HIDDEN] / cnt
    h = jnp.maximum(
        jnp.dot(mean, w1t[...], preferred_element_type=jnp.float32) + b1[...], 0.0)
    o[...] = jnp.dot(h, w2t[...], preferred_element_type=jnp.float32) + b2[...]


def _mlp(p, w1t, b1, w2t, b2):
    return pl.pallas_call(
        _mlp_body,
        out_shape=jax.ShapeDtypeStruct((NUM_GRAPHS, 2), jnp.float32),
    )(p, w1t, b1, w2t, b2)


def kernel(x, edge_index, edge_attr, u, batch, W1, b1, W2, b2):
    xp = jnp.concatenate(
        [x, jnp.ones((N_NODES, 1), jnp.float32),
         jnp.zeros((N_NODES, 1), jnp.float32)], axis=1)
    idx = jnp.pad(batch.astype(jnp.int32), (0, N_PAD - N_NODES),
                  constant_values=NUM_GRAPHS)
    z = jnp.zeros((ROWS_PER_TILE, W), jnp.float32)
    p = _seg_kernel(xp, idx, z)
    return _mlp(p, W1.T, b1[None, :], W2.T, b2[None, :])
